# baseline (speedup 1.0000x reference)
import jax
import jax.numpy as jnp
from jax import lax
from jax.experimental import pallas as pl
from jax.experimental.pallas import tpu as pltpu
from jax.experimental.pallas import tpu_sc as plsc

_DIM = 64
_B = 4096
_CTX = 10
_NS = 21
_NC, _NSUB = 2, 16
_NW = _NC * _NSUB
_BW = _B // _NW
_YW = _BW * _NS
_CW = _BW * _CTX
_CB = 16
_NCHUNK = _BW // _CB
_CROWS = _CB * _NS


def _unpack_row(ref, r, q):
    return plsc.unpack(ref[r, pl.ds(q * 32, 32)],
                       format=plsc.PackFormat.INTERLEAVED)


def _dmm_body(ctxs_ref, docs_ref, y_ref, d_ref, w_ref, ot_ref, out_ref,
              docs_v, ctx_v, y_v, docb, ctxrows, xbuf, col0, col1, scores_v,
              sem0, sem1):
    wid = lax.axis_index("s") * _NC + lax.axis_index("c")

    pltpu.sync_copy(docs_ref.at[pl.ds(wid * _BW, _BW)], docs_v)
    pltpu.sync_copy(ctxs_ref.at[pl.ds(wid * _CW, _CW)], ctx_v)
    pltpu.sync_copy(y_ref.at[pl.ds(wid * _YW, _YW)], y_v.at[pl.ds(0, _YW)])

    descs = [None, None]
    descs[0] = pltpu.async_copy(ot_ref.at[y_v.at[pl.ds(0, _CROWS)]], col0, sem0)

    pltpu.sync_copy(d_ref.at[docs_v], docb)
    half = _CW // 2
    cols = [col0, col1]
    sems = [sem0, sem1]

    for h in range(2):
        pltpu.sync_copy(w_ref.at[ctx_v.at[pl.ds(h * half, half)]], ctxrows)

        def x_body(b, carry, h=h):
            gb = h * (_BW // 2) + b
            for q in range(2):
                xa, xb = _unpack_row(docb, gb, q)
                for c in range(_CTX):
                    ca, cb = _unpack_row(ctxrows, b * _CTX + c, q)
                    xa = xa + ca
                    xb = xb + cb
                xbuf[gb, pl.ds(q * 32, 16)] = xa
                xbuf[gb, pl.ds(q * 32 + 16, 16)] = xb
            return carry
        lax.fori_loop(0, _BW // 2, x_body, 0)

    lanes = lax.iota(jnp.int32, 16)
    for c in range(_NCHUNK):
        if c + 1 < _NCHUNK:
            nsel = (c + 1) % 2
            descs[nsel] = pltpu.async_copy(
                ot_ref.at[y_v.at[pl.ds((c + 1) * _CROWS, _CROWS)]],
                cols[nsel], sems[nsel])
        sel = c % 2
        descs[sel].wait()
        colbuf = cols[sel]

        def dot_body(lb, carry, c=c, colbuf=colbuf):
            b = c * _CB + lb
            xq = [xbuf[b, pl.ds(k * 16, 16)] for k in range(4)]
            ss = []
            for n in range(_NS):
                r = lb * _NS + n
                oa0, ob0 = _unpack_row(colbuf, r, 0)
                oa1, ob1 = _unpack_row(colbuf, r, 1)
                v = xq[0] * oa0 + xq[1] * ob0 + xq[2] * oa1 + xq[3] * ob1
                ss.append(jnp.sum(v))
            vec0 = jnp.broadcast_to(ss[0], (16,))
            for n in range(1, 16):
                vec0 = jnp.where(lanes == n, ss[n], vec0)
            vec1 = jnp.broadcast_to(ss[16], (16,))
            for n in range(17, _NS):
                vec1 = jnp.where(lanes == (n - 16), ss[n], vec1)
            scores_v[pl.ds(b * _NS, 16)] = vec0
            scores_v[pl.ds(b * _NS + 16, 16)] = vec1
            return carry
        lax.fori_loop(0, _CB, dot_body, 0)

    pltpu.sync_copy(scores_v.at[pl.ds(0, _YW)], out_ref.at[pl.ds(wid * _YW, _YW)])


def kernel(ctxs, docs, y, D, W, O):
    ctxs_f = ctxs.reshape(-1).astype(jnp.int32)
    docs_i = docs.reshape(-1).astype(jnp.int32)
    y_f = y.reshape(-1).astype(jnp.int32)
    db = D.astype(jnp.bfloat16)
    wb = W.astype(jnp.bfloat16)
    ot = O.astype(jnp.bfloat16).T
    run = pl.kernel(
        _dmm_body,
        out_type=jax.ShapeDtypeStruct((_B * _NS,), jnp.float32),
        mesh=plsc.VectorSubcoreMesh(
            core_axis_name="c", subcore_axis_name="s",
            num_cores=_NC, num_subcores=_NSUB),
        compiler_params=pltpu.CompilerParams(
            use_tc_tiling_on_sc=False, needs_layout_passes=False),
        scratch_types=[
            pltpu.VMEM((_BW,), jnp.int32),
            pltpu.VMEM((_CW,), jnp.int32),
            pltpu.VMEM((_YW + 16,), jnp.int32),
            pltpu.VMEM((_BW, _DIM), jnp.bfloat16),
            pltpu.VMEM((_CW // 2, _DIM), jnp.bfloat16),
            pltpu.VMEM((_BW, _DIM), jnp.float32),
            pltpu.VMEM((_CROWS, _DIM), jnp.bfloat16),
            pltpu.VMEM((_CROWS, _DIM), jnp.bfloat16),
            pltpu.VMEM((_YW + 16,), jnp.float32),
            pltpu.SemaphoreType.DMA,
            pltpu.SemaphoreType.DMA,
        ],
    )
    return run(ctxs_f, docs_i, y_f, db, wb, ot).reshape(_B, _NS)


# speedup vs baseline: 1.1740x; 1.1740x over previous
import jax
import jax.numpy as jnp
from jax import lax
from jax.experimental import pallas as pl
from jax.experimental.pallas import tpu as pltpu
from jax.experimental.pallas import tpu_sc as plsc

_DIM = 64
_B = 4096
_CTX = 10
_NS = 21
_NC, _NSUB = 2, 16
_NW = _NC * _NSUB
_BW = _B // _NW
_YW = _BW * _NS
_CW = _BW * _CTX
_CB = 16
_NCHUNK = _BW // _CB
_CROWS = _CB * _NS

_MESH = dict(
    mesh=plsc.VectorSubcoreMesh(
        core_axis_name="c", subcore_axis_name="s",
        num_cores=_NC, num_subcores=_NSUB),
    compiler_params=pltpu.CompilerParams(
        use_tc_tiling_on_sc=False, needs_layout_passes=False),
)


def _x_body(ctxs_ref, docs_ref, d_ref, w_ref, x_ref,
            docs_v, ctx_v, docrows, ctxrows):
    wid = lax.axis_index("s") * _NC + lax.axis_index("c")
    pltpu.sync_copy(docs_ref.at[pl.ds(wid * _BW, _BW)], docs_v)
    pltpu.sync_copy(ctxs_ref.at[pl.ds(wid * _CW, _CW)], ctx_v)
    pltpu.sync_copy(d_ref.at[docs_v], docrows)
    half = _CW // 2
    for h in range(2):
        pltpu.sync_copy(w_ref.at[ctx_v.at[pl.ds(h * half, half)]], ctxrows)

        def body(b, carry, h=h):
            gb = h * (_BW // 2) + b
            for q in range(_DIM // 16):
                acc0 = docrows[gb, pl.ds(q * 16, 16)]
                acc1 = ctxrows[b * _CTX, pl.ds(q * 16, 16)]
                for c in range(1, _CTX, 2):
                    acc0 = acc0 + ctxrows[b * _CTX + c, pl.ds(q * 16, 16)]
                    if c + 1 < _CTX:
                        acc1 = acc1 + ctxrows[b * _CTX + c + 1, pl.ds(q * 16, 16)]
                docrows[gb, pl.ds(q * 16, 16)] = acc0 + acc1
            return carry
        lax.fori_loop(0, _BW // 2, body, 0)
    pltpu.sync_copy(docrows, x_ref.at[pl.ds(wid * _BW, _BW)])


def _score_body(y_ref, x_ref, ot_ref, out_ref,
                y_v, xloc, col0, col1, scores_v, sem0, sem1):
    wid = lax.axis_index("s") * _NC + lax.axis_index("c")
    pltpu.sync_copy(y_ref.at[pl.ds(wid * _YW, _YW)], y_v.at[pl.ds(0, _YW)])
    descs = [None, None]
    descs[0] = pltpu.async_copy(ot_ref.at[y_v.at[pl.ds(0, _CROWS)]], col0, sem0)
    pltpu.sync_copy(x_ref.at[pl.ds(wid * _BW, _BW)], xloc)
    cols = [col0, col1]
    sems = [sem0, sem1]
    lanes = lax.iota(jnp.int32, 16)
    for c in range(_NCHUNK):
        if c + 1 < _NCHUNK:
            nsel = (c + 1) % 2
            descs[nsel] = pltpu.async_copy(
                ot_ref.at[y_v.at[pl.ds((c + 1) * _CROWS, _CROWS)]],
                cols[nsel], sems[nsel])
        sel = c % 2
        descs[sel].wait()
        colbuf = cols[sel]

        def dot_body(g, carry, c=c, colbuf=colbuf):
            rvec = lanes + g * 16
            bvec = (rvec + c * _CROWS) // _NS
            acc0 = jnp.zeros((16,), jnp.float32)
            acc1 = jnp.zeros((16,), jnp.float32)
            for d in range(_DIM):
                dsplat = jnp.full((16,), d, jnp.int32)
                ov = plsc.load_gather(colbuf, [rvec, dsplat])
                xv = plsc.load_gather(xloc, [bvec, dsplat])
                if d % 2 == 0:
                    acc0 = acc0 + xv * ov
                else:
                    acc1 = acc1 + xv * ov
            scores_v[pl.ds(c * _CROWS + g * 16, 16)] = acc0 + acc1
            return carry
        lax.fori_loop(0, _CROWS // 16, dot_body, 0)

    pltpu.sync_copy(scores_v.at[pl.ds(0, _YW)], out_ref.at[pl.ds(wid * _YW, _YW)])


def kernel(ctxs, docs, y, D, W, O):
    ctxs_f = ctxs.reshape(-1).astype(jnp.int32)
    docs_i = docs.reshape(-1).astype(jnp.int32)
    y_f = y.reshape(-1).astype(jnp.int32)
    ot = O.T
    k1 = pl.kernel(
        _x_body,
        out_type=jax.ShapeDtypeStruct((_B, _DIM), jnp.float32),
        scratch_types=[
            pltpu.VMEM((_BW,), jnp.int32),
            pltpu.VMEM((_CW,), jnp.int32),
            pltpu.VMEM((_BW, _DIM), jnp.float32),
            pltpu.VMEM((_CW // 2, _DIM), jnp.float32),
        ],
        **_MESH,
    )
    x = k1(ctxs_f, docs_i, D, W)
    k2 = pl.kernel(
        _score_body,
        out_type=jax.ShapeDtypeStruct((_B * _NS,), jnp.float32),
        scratch_types=[
            pltpu.VMEM((_YW + 16,), jnp.int32),
            pltpu.VMEM((_BW, _DIM), jnp.float32),
            pltpu.VMEM((_CROWS, _DIM), jnp.float32),
            pltpu.VMEM((_CROWS, _DIM), jnp.float32),
            pltpu.VMEM((_YW + 16,), jnp.float32),
            pltpu.SemaphoreType.DMA,
            pltpu.SemaphoreType.DMA,
        ],
        **_MESH,
    )
    return k2(y_f, x, ot).reshape(_B, _NS)


# speedup vs baseline: 1.2334x; 1.0506x over previous
import jax
import jax.numpy as jnp
from jax import lax
from jax.experimental import pallas as pl
from jax.experimental.pallas import tpu as pltpu
from jax.experimental.pallas import tpu_sc as plsc

_DIM = 64
_B = 4096
_CTX = 10
_NS = 21
_NC, _NSUB = 2, 16
_NW = _NC * _NSUB
_BW = _B // _NW
_YW = _BW * _NS
_CW = _BW * _CTX
_CB = 16
_NCHUNK = _BW // _CB
_CROWS = _CB * _NS
_PADW = 128

_MESH = dict(
    mesh=plsc.VectorSubcoreMesh(
        core_axis_name="c", subcore_axis_name="s",
        num_cores=_NC, num_subcores=_NSUB),
    compiler_params=pltpu.CompilerParams(
        use_tc_tiling_on_sc=False, needs_layout_passes=False),
)


def _x_body(ctxs_ref, docs_ref, d_ref, w_ref, x_ref,
            docs_v, ctx_v, docrows, ctxrows):
    wid = lax.axis_index("s") * _NC + lax.axis_index("c")
    pltpu.sync_copy(docs_ref.at[pl.ds(wid * _BW, _BW)], docs_v)
    pltpu.sync_copy(ctxs_ref.at[pl.ds(wid * _CW, _CW)], ctx_v)
    pltpu.sync_copy(d_ref.at[docs_v], docrows)
    half = _CW // 2
    for h in range(2):
        pltpu.sync_copy(w_ref.at[ctx_v.at[pl.ds(h * half, half)]], ctxrows)

        def body(b, carry, h=h):
            gb = h * (_BW // 2) + b
            for q in range(_DIM // 16):
                acc0 = docrows[gb, pl.ds(q * 16, 16)]
                acc1 = ctxrows[b * _CTX, pl.ds(q * 16, 16)]
                for c in range(1, _CTX, 2):
                    acc0 = acc0 + ctxrows[b * _CTX + c, pl.ds(q * 16, 16)]
                    if c + 1 < _CTX:
                        acc1 = acc1 + ctxrows[b * _CTX + c + 1, pl.ds(q * 16, 16)]
                docrows[gb, pl.ds(q * 16, 16)] = acc0 + acc1
            return carry
        lax.fori_loop(0, _BW // 2, body, 0)
    pltpu.sync_copy(docrows, x_ref.at[pl.ds(wid * _BW, _BW)])


def _score_body(y_ref, x_ref, ot_ref, out_ref,
                y_v, xloc, col0, col1, scores_v, sem0, sem1):
    wid = lax.axis_index("s") * _NC + lax.axis_index("c")
    pltpu.sync_copy(y_ref.at[pl.ds(wid * _YW, _YW)], y_v.at[pl.ds(0, _YW)])
    descs = [None, None]
    descs[0] = pltpu.async_copy(ot_ref.at[y_v.at[pl.ds(0, _CROWS)]], col0, sem0)
    pltpu.sync_copy(x_ref.at[pl.ds(wid * _BW, _BW)], xloc)
    cols = [col0, col1]
    sems = [sem0, sem1]
    lanes = lax.iota(jnp.int32, 16)
    for c in range(_NCHUNK):
        if c + 1 < _NCHUNK:
            nsel = (c + 1) % 2
            descs[nsel] = pltpu.async_copy(
                ot_ref.at[y_v.at[pl.ds((c + 1) * _CROWS, _CROWS)]],
                cols[nsel], sems[nsel])
        sel = c % 2
        descs[sel].wait()
        colbuf = cols[sel]

        def dot_body(g, carry, c=c, colbuf=colbuf):
            rvec = lanes + g * 16
            bvec = (rvec + c * _CROWS) // _NS
            acc0 = jnp.zeros((16,), jnp.float32)
            acc1 = jnp.zeros((16,), jnp.float32)
            for d in range(_DIM):
                dsplat = jnp.full((16,), d, jnp.int32)
                ov = plsc.load_gather(colbuf, [rvec, dsplat])
                xv = plsc.load_gather(xloc, [bvec, dsplat])
                if d % 2 == 0:
                    acc0 = acc0 + xv * ov
                else:
                    acc1 = acc1 + xv * ov
            scores_v[pl.ds(c * _CROWS + g * 16, 16)] = acc0 + acc1
            return carry
        lax.fori_loop(0, _CROWS // 16, dot_body, 0)

    pltpu.sync_copy(scores_v.at[pl.ds(0, _YW)], out_ref.at[pl.ds(wid * _YW, _YW)])


def kernel(ctxs, docs, y, D, W, O):
    ctxs_f = ctxs.reshape(-1).astype(jnp.int32)
    docs_i = docs.reshape(-1).astype(jnp.int32)
    y_f = y.reshape(-1).astype(jnp.int32)
    ot = jnp.pad(O, ((0, _PADW - _DIM), (0, 0))).T
    wp = jnp.pad(W, ((0, 0), (0, _PADW - _DIM)))
    k1 = pl.kernel(
        _x_body,
        out_type=jax.ShapeDtypeStruct((_B, _DIM), jnp.float32),
        scratch_types=[
            pltpu.VMEM((_BW,), jnp.int32),
            pltpu.VMEM((_CW,), jnp.int32),
            pltpu.VMEM((_BW, _DIM), jnp.float32),
            pltpu.VMEM((_CW // 2, _PADW), jnp.float32),
        ],
        **_MESH,
    )
    x = k1(ctxs_f, docs_i, D, wp)
    k2 = pl.kernel(
        _score_body,
        out_type=jax.ShapeDtypeStruct((_B * _NS,), jnp.float32),
        scratch_types=[
            pltpu.VMEM((_YW + 16,), jnp.int32),
            pltpu.VMEM((_BW, _DIM), jnp.float32),
            pltpu.VMEM((_CROWS, _PADW), jnp.float32),
            pltpu.VMEM((_CROWS, _PADW), jnp.float32),
            pltpu.VMEM((_YW + 16,), jnp.float32),
            pltpu.SemaphoreType.DMA,
            pltpu.SemaphoreType.DMA,
        ],
        **_MESH,
    )
    return k2(y_f, x, ot).reshape(_B, _NS)
